# Initial kernel scaffold; baseline (speedup 1.0000x reference)
import jax
import jax.numpy as jnp
from jax.experimental import pallas as pl


def kernel(x, edge_index, batch, W1, b1, p1, W2, b2, p2, Wfc, bfc):
    raise NotImplementedError("write your pallas kernel here")


# speedup vs baseline: 15.9583x; 15.9583x over previous
import functools
import math

import jax
import jax.numpy as jnp
from jax import lax
from jax.experimental import pallas as pl
from jax.experimental.pallas import tpu as pltpu
from jax.experimental.pallas import tpu_sc as plsc

_NC = 2
_NS = 16


def _largest_div(n, cap, mult=8):
    best = None
    for d in range(1, n + 1):
        if n % d == 0 and d <= cap and d % mult == 0:
            best = d
    return best


@functools.cache
def _sc_edge_pass(n_nodes, n_edges, width, with_gather):
    nw = _NC * _NS
    ept = n_edges // nw
    assert ept * nw == n_edges
    ch = _largest_div(ept, 128)
    npad = -(-n_nodes // 2048) * 2048
    rpt = npad // _NS
    zr = _largest_div(rpt, max(8, 65536 // (width * 4)))

    mesh = plsc.VectorSubcoreMesh(
        core_axis_name="c", subcore_axis_name="s",
        num_cores=_NC, num_subcores=_NS)

    scratch = [
        pltpu.VMEM((ch,), jnp.int32),
        pltpu.VMEM((ch, width), jnp.float32),
        pltpu.VMEM((zr, width), jnp.float32),
        pltpu.VMEM_SHARED((npad, width), jnp.float32),
        pltpu.SemaphoreType.DMA,
    ]
    if with_gather:
        scratch.insert(0, pltpu.VMEM((ch,), jnp.int32))

    def _fill(ref, rows, value):
        def frow(i, _):
            def fcol(j, _):
                ref[i, pl.ds(j * 16, 16)] = jnp.full((16,), value, jnp.float32)
                return 0
            lax.fori_loop(0, width // 16, fcol, 0)
            return 0
        lax.fori_loop(0, rows, frow, 0)

    def body(*refs):
        if with_gather:
            (table_hbm, src_hbm, dst_hbm, out_hbm,
             src_v, dst_v, rows_v, zbuf, acc, sem) = refs
        else:
            (dst_hbm, out_hbm, dst_v, rows_v, zbuf, acc, sem) = refs
        c = lax.axis_index("c")
        s = lax.axis_index("s")
        _fill(zbuf, zr, 0.0)
        row0 = s * rpt
        for t in range(rpt // zr):
            pltpu.sync_copy(zbuf, acc.at[pl.ds(row0 + t * zr, zr)])
        if not with_gather:
            _fill(rows_v, ch, 1.0)
        plsc.subcore_barrier()
        base = (c * _NS + s) * ept
        def chunk(i, _):
            off = base + i * ch
            pltpu.sync_copy(dst_hbm.at[pl.ds(off, ch)], dst_v)
            if with_gather:
                pltpu.sync_copy(src_hbm.at[pl.ds(off, ch)], src_v)
                pltpu.async_copy(table_hbm.at[src_v], rows_v, sem).wait()
            pltpu.sync_copy(rows_v, acc.at[dst_v], add=True)
            return 0
        lax.fori_loop(0, ept // ch, chunk, 0)
        plsc.subcore_barrier()
        for t in range(rpt // zr):
            r = row0 + t * zr
            pltpu.sync_copy(acc.at[pl.ds(r, zr)], out_hbm.at[c, pl.ds(r, zr)])

    return pl.kernel(
        body,
        out_type=jax.ShapeDtypeStruct((_NC, npad, width), jnp.float32),
        mesh=mesh,
        scratch_types=scratch,
        compiler_params=pltpu.CompilerParams(use_tc_tiling_on_sc=False),
    )


_RB = 1000


def _prescale_body(hist_ref, feat_ref, xs_ref, dinv_ref):
    h = hist_ref[...]
    cnt = h[0, :, 0:1] + h[1, :, 0:1]
    d = lax.rsqrt(cnt + 1.0)
    xs_ref[...] = feat_ref[...] * d
    dinv_ref[...] = d


def _prescale(hist, feat):
    n, f = feat.shape
    return pl.pallas_call(
        _prescale_body,
        grid=(n // _RB,),
        in_specs=[
            pl.BlockSpec((_NC, _RB, 16), lambda i: (0, i, 0)),
            pl.BlockSpec((_RB, f), lambda i: (i, 0)),
        ],
        out_specs=[
            pl.BlockSpec((_RB, f), lambda i: (i, 0)),
            pl.BlockSpec((_RB, 1), lambda i: (i, 0)),
        ],
        out_shape=[
            jax.ShapeDtypeStruct((n, f), jnp.float32),
            jax.ShapeDtypeStruct((n, 1), jnp.float32),
        ],
    )(hist, feat)


def _conv_mm_body(u_ref, x_ref, d_ref, w_ref, b_ref, p_ref, h_ref, s_ref):
    d = d_ref[...]
    z = d * (u_ref[0] + u_ref[1]) + (d * d) * x_ref[...]
    h = jnp.dot(z, w_ref[...], preferred_element_type=jnp.float32) + b_ref[...]
    h = jnp.maximum(h, 0.0)
    h_ref[...] = h
    p = p_ref[...]
    s_ref[...] = jnp.dot(h, p, preferred_element_type=jnp.float32) * lax.rsqrt(
        jnp.sum(p * p))


def _conv_mm(u, x, dinv, W, b, p):
    n, f = x.shape
    fo = W.shape[1]
    return pl.pallas_call(
        _conv_mm_body,
        grid=(n // _RB,),
        in_specs=[
            pl.BlockSpec((_NC, _RB, f), lambda i: (0, i, 0)),
            pl.BlockSpec((_RB, f), lambda i: (i, 0)),
            pl.BlockSpec((_RB, 1), lambda i: (i, 0)),
            pl.BlockSpec((f, fo), lambda i: (0, 0)),
            pl.BlockSpec((1, fo), lambda i: (0, 0)),
            pl.BlockSpec((fo, 1), lambda i: (0, 0)),
        ],
        out_specs=[
            pl.BlockSpec((_RB, fo), lambda i: (i, 0)),
            pl.BlockSpec((_RB, 1), lambda i: (i, 0)),
        ],
        out_shape=[
            jax.ShapeDtypeStruct((n, fo), jnp.float32),
            jax.ShapeDtypeStruct((n, 1), jnp.float32),
        ],
    )(u, x, dinv, W, b, p)


def _float_key(s):
    ub = lax.bitcast_convert_type(s, jnp.uint32)
    top = jnp.uint32(0x80000000)
    return jnp.where(ub >= top, ~ub, ub ^ top)


def _topk_mask(key, k, n):
    def bs(_, lohi):
        lo, hi = lohi
        mid = lo + ((hi - lo) >> jnp.uint32(1)) + jnp.uint32(1)
        ge = jnp.sum((key >= mid).astype(jnp.int32)) >= k
        return (jnp.where(ge, mid, lo), jnp.where(ge, hi, mid - jnp.uint32(1)))
    t, _ = lax.fori_loop(0, 32, bs, (jnp.uint32(0), jnp.uint32(0xFFFFFFFF)))
    cgt = jnp.sum((key > t).astype(jnp.int32))
    r = k - cgt
    idx = lax.broadcasted_iota(jnp.int32, key.shape, 0)
    tie = key == t
    def bs2(_, lohi):
        lo, hi = lohi
        mid = (lo + hi) >> 1
        ge = jnp.sum((tie & (idx <= mid)).astype(jnp.int32)) >= r
        return (jnp.where(ge, lo, mid + 1), jnp.where(ge, mid, hi))
    j, _ = lax.fori_loop(0, 14, bs2, (jnp.int32(0), jnp.int32(n - 1)))
    return ((key > t) | (tie & (idx <= j))).astype(jnp.float32)


def _topk1_body(k, n, s_ref, m_ref, mrep_ref):
    m = _topk_mask(_float_key(s_ref[...]), k, n)
    m_ref[...] = m
    mrep_ref[...] = jnp.broadcast_to(m, (n, 16))


def _topk1(score, k):
    n = score.shape[0]
    return pl.pallas_call(
        functools.partial(_topk1_body, k, n),
        out_shape=[
            jax.ShapeDtypeStruct((n, 1), jnp.float32),
            jax.ShapeDtypeStruct((n, 16), jnp.float32),
        ],
    )(score)


def _pool_mm_body(h_ref, s_ref, m_ref, w_ref, o_ref):
    hp = h_ref[...] * (jnp.tanh(s_ref[...]) * m_ref[...])
    o_ref[...] = jnp.dot(hp, w_ref[...], preferred_element_type=jnp.float32)


def _pool_mm(h, score, m, W):
    n, f = h.shape
    fo = W.shape[1]
    return pl.pallas_call(
        _pool_mm_body,
        grid=(n // _RB,),
        in_specs=[
            pl.BlockSpec((_RB, f), lambda i: (i, 0)),
            pl.BlockSpec((_RB, 1), lambda i: (i, 0)),
            pl.BlockSpec((_RB, 1), lambda i: (i, 0)),
            pl.BlockSpec((f, fo), lambda i: (0, 0)),
        ],
        out_specs=pl.BlockSpec((_RB, fo), lambda i: (i, 0)),
        out_shape=jax.ShapeDtypeStruct((n, fo), jnp.float32),
    )(h, score, m, W)


def _conv2_post_body(u_ref, h_ref, d_ref, b_ref, p_ref, m_ref, o_ref, k_ref):
    d = d_ref[...]
    o = d * (u_ref[0] + u_ref[1]) + (d * d) * h_ref[...] + b_ref[...]
    o = jnp.maximum(o, 0.0)
    o_ref[...] = o
    p = p_ref[...]
    s = jnp.dot(o, p, preferred_element_type=jnp.float32) * lax.rsqrt(
        jnp.sum(p * p))
    k_ref[...] = jnp.where(m_ref[...] > 0.0, s, -jnp.inf)


def _conv2_post(u, h, dinv, b, p, m):
    n, f = h.shape
    return pl.pallas_call(
        _conv2_post_body,
        grid=(n // _RB,),
        in_specs=[
            pl.BlockSpec((_NC, _RB, f), lambda i: (0, i, 0)),
            pl.BlockSpec((_RB, f), lambda i: (i, 0)),
            pl.BlockSpec((_RB, 1), lambda i: (i, 0)),
            pl.BlockSpec((1, f), lambda i: (0, 0)),
            pl.BlockSpec((f, 1), lambda i: (0, 0)),
            pl.BlockSpec((_RB, 1), lambda i: (i, 0)),
        ],
        out_specs=[
            pl.BlockSpec((_RB, f), lambda i: (i, 0)),
            pl.BlockSpec((_RB, 1), lambda i: (i, 0)),
        ],
        out_shape=[
            jax.ShapeDtypeStruct((n, f), jnp.float32),
            jax.ShapeDtypeStruct((n, 1), jnp.float32),
        ],
    )(u, h, dinv, b, p, m)


def _final_body(k, n, o_ref, key_ref, w_ref, b_ref, out_ref):
    s = key_ref[...]
    m2 = _topk_mask(_float_key(s), k, n)
    w = jnp.tanh(s) * m2 * (1.0 / k)
    g = jnp.sum(o_ref[...] * w, axis=0, keepdims=True)
    logits = jnp.dot(g, w_ref[...], preferred_element_type=jnp.float32) \
        + b_ref[...]
    valid = lax.broadcasted_iota(jnp.int32, logits.shape, 1) < 3
    neg = jnp.float32(-3.0e38)
    lm = jnp.where(valid, logits, neg)
    mx = jnp.max(lm)
    lse = jnp.log(jnp.sum(jnp.where(valid, jnp.exp(logits - mx), 0.0)))
    out_ref[...] = logits - mx - lse


def _final(out2, key2, Wp, bp, k):
    n, f = out2.shape
    return pl.pallas_call(
        functools.partial(_final_body, k, n),
        out_shape=jax.ShapeDtypeStruct((1, 128), jnp.float32),
    )(out2, key2, Wp, bp)


def kernel(x, edge_index, batch, W1, b1, p1, W2, b2, p2, Wfc, bfc):
    n, f = x.shape
    e = edge_index.shape[1]
    src = edge_index[0]
    dst = edge_index[1]
    k1 = int(math.ceil(0.8 * n))
    k2 = int(math.ceil(0.8 * k1))

    hist1 = _sc_edge_pass(n, e, 16, False)(dst)
    xs1, dinv1 = _prescale(hist1, x)
    u1 = _sc_edge_pass(n, e, f, True)(xs1, src, dst)
    h, s1 = _conv_mm(u1, x, dinv1, W1, b1.reshape(1, -1), p1.reshape(-1, 1))

    m1, m1rep = _topk1(s1, k1)
    h2 = _pool_mm(h, s1, m1, W2)

    hist2 = _sc_edge_pass(n, e, 16, True)(m1rep, src, dst)
    xs2, dinv2 = _prescale(hist2, h2)
    u2 = _sc_edge_pass(n, e, f, True)(xs2, src, dst)
    out2, key2 = _conv2_post(u2, h2, dinv2, b2.reshape(1, -1),
                             p2.reshape(-1, 1), m1)

    Wp = jnp.pad(Wfc, ((0, 0), (0, 128 - Wfc.shape[1])))
    bp = jnp.pad(bfc, (0, 128 - bfc.shape[0])).reshape(1, -1)
    res = _final(out2, key2, Wp, bp, k2)
    return res[:, :3]
